# Initial kernel scaffold; baseline (speedup 1.0000x reference)
import jax
import jax.numpy as jnp
from jax.experimental import pallas as pl


def kernel(feat, W_router, W_shared, W_experts):
    raise NotImplementedError("write your pallas kernel here")


# speedup vs baseline: 2.9317x; 2.9317x over previous
import functools

import jax
import jax.numpy as jnp
from jax.experimental import pallas as pl
from jax.experimental.pallas import tpu as pltpu


def _moe_dense_body(x_ref, wr_ref, we_ref, ws_ref, out_ref, aux_ref, acc_ref,
                    *, n_tokens: int, n_experts: int):
    i = pl.program_id(0)
    nb = pl.num_programs(0)
    x = x_ref[...]

    logits = jax.lax.dot_general(
        x, wr_ref[...], (((1,), (1,)), ((), ())),
        preferred_element_type=jnp.float32)

    e_iota = jax.lax.broadcasted_iota(jnp.int32, logits.shape, 1)
    m1 = jnp.max(logits, axis=-1, keepdims=True)
    i1 = jnp.min(jnp.where(logits == m1, e_iota, n_experts), axis=-1,
                 keepdims=True)
    oh1 = (e_iota == i1).astype(jnp.float32)
    masked = jnp.where(e_iota == i1, -jnp.inf, logits)
    m2 = jnp.max(masked, axis=-1, keepdims=True)
    i2 = jnp.min(jnp.where(masked == m2, e_iota, n_experts), axis=-1,
                 keepdims=True)
    oh2 = (e_iota == i2).astype(jnp.float32)
    w2 = 1.0 / (1.0 + jnp.exp(m1 - m2))
    w1 = 1.0 - w2
    gate = w1 * oh1 + w2 * oh2

    z = jnp.exp(logits - m1)
    sc = z / jnp.sum(z, axis=-1, keepdims=True)

    @pl.when(i == 0)
    def _init():
        acc_ref[...] = jnp.zeros_like(acc_ref)

    acc_ref[0, :] += jnp.sum(sc, axis=0)
    acc_ref[1, :] += jnp.sum(oh1 + oh2, axis=0)

    xb = x.astype(jnp.bfloat16)
    acc = jnp.zeros(out_ref.shape, jnp.float32)
    for e in range(n_experts):
        ye = jax.lax.dot_general(
            xb, we_ref[e], (((1,), (1,)), ((), ())),
            preferred_element_type=jnp.float32)
        acc += gate[:, e:e + 1] * ye
    ws = (ws_ref[0].astype(jnp.float32)
          + ws_ref[1].astype(jnp.float32)).astype(jnp.bfloat16)
    acc += jax.lax.dot_general(
        xb, ws, (((1,), (1,)), ((), ())), preferred_element_type=jnp.float32)
    out_ref[...] = acc

    @pl.when(i == nb - 1)
    def _fin():
        pi = acc_ref[0, :] / n_tokens
        fi = acc_ref[1, :] / n_tokens
        aux_ref[...] = jnp.sum(pi * fi).reshape(1, 1)


def kernel(feat, W_router, W_shared, W_experts):
    B, S, d = feat.shape
    N = B * S
    E = W_router.shape[0]
    x = feat.reshape(N, d)
    we = W_experts.astype(jnp.bfloat16)
    ws = W_shared.reshape(-1, d, d).astype(jnp.bfloat16)
    n_shared = ws.shape[0]
    assert n_shared == 2
    BLK = 512
    nb = N // BLK
    out, aux = pl.pallas_call(
        functools.partial(_moe_dense_body, n_tokens=N, n_experts=E),
        grid=(nb,),
        in_specs=[
            pl.BlockSpec((BLK, d), lambda i: (i, 0)),
            pl.BlockSpec((E, d), lambda i: (0, 0)),
            pl.BlockSpec((E, d, d), lambda i: (0, 0, 0)),
            pl.BlockSpec((n_shared, d, d), lambda i: (0, 0, 0)),
        ],
        out_specs=[
            pl.BlockSpec((BLK, d), lambda i: (i, 0)),
            pl.BlockSpec((1, 1), lambda i: (0, 0)),
        ],
        out_shape=[
            jax.ShapeDtypeStruct((N, d), jnp.float32),
            jax.ShapeDtypeStruct((1, 1), jnp.float32),
        ],
        scratch_shapes=[pltpu.VMEM((2, E), jnp.float32)],
    )(x, W_router, we, ws)
    return out.reshape(B, S, d), aux[0, 0]
